# Initial kernel scaffold; baseline (speedup 1.0000x reference)
import jax
import jax.numpy as jnp
from jax.experimental import pallas as pl


def kernel(x, aug_feats, edge_index_1, edge_index_2, W_proj, b_proj, W_att, b_att, q_att):
    raise NotImplementedError("write your pallas kernel here")


# speedup vs baseline: 13.9259x; 13.9259x over previous
import functools

import jax
import jax.numpy as jnp
from jax import lax
from jax.experimental import pallas as pl
from jax.experimental.pallas import tpu as pltpu
from jax.experimental.pallas import tpu_sc as plsc

N = 10000
E = 320000
D_FEAT = 128
EMB = 64
LABELS = 16
ATT = 128
K_LAYER = 3
ALPHA = 0.1

NC = 2
NS = 16
L = 16

CHUNK = 128
CPT = 160
E_PAD = CPT * CHUNK * NS
N_PAD = 10240
RPT = N_PAD // NS
DUMP = N_PAD


def _proj_body(x_ref, aug_ref, wt_ref, wb_ref, b_ref, o_ref):
    temp = (aug_ref[0] + aug_ref[1] + aug_ref[2] + aug_ref[3]) * 0.25
    z = jnp.dot(x_ref[...], wt_ref[...], preferred_element_type=jnp.float32)
    z = z + jnp.dot(temp, wb_ref[...], preferred_element_type=jnp.float32)
    z = z + b_ref[...]
    o_ref[pl.ds(0, N)] = jnp.maximum(z, 0.0)
    o_ref[pl.ds(N, N_PAD - N)] = jnp.zeros((N_PAD - N, LABELS), jnp.float32)


def _project(x, aug_feats, W_proj, b_proj):
    return pl.pallas_call(
        _proj_body,
        out_shape=jax.ShapeDtypeStruct((N_PAD, LABELS), jnp.float32),
    )(x, aug_feats, W_proj[:D_FEAT], W_proj[D_FEAT:], b_proj.reshape(1, LABELS))


def _prop_body(z0_hbm, srcs_hbm, dsts_hbm, out_hbm,
               idx_s, idx_d, rb0, rb1, ones_v, A, D, Z, zeros_v,
               agg_s, deg_s, sem0, sem1):
    c = lax.axis_index("c")
    s = lax.axis_index("s")
    rowbase = s * RPT

    pltpu.sync_copy(srcs_hbm.at[c, pl.ds(s * CPT, CPT)], idx_s)
    pltpu.sync_copy(dsts_hbm.at[c, pl.ds(s * CPT, CPT)], idx_d)

    @pl.loop(0, CHUNK)
    def _(i):
        ones_v[i] = jnp.full((L,), 1.0, jnp.float32)

    @pl.loop(0, RPT)
    def _(i):
        zeros_v[i] = jnp.zeros((L,), jnp.float32)

    pltpu.sync_copy(z0_hbm.at[pl.ds(rowbase, RPT)], Z)

    pltpu.sync_copy(zeros_v, agg_s.at[pl.ds(rowbase, RPT)])
    pltpu.sync_copy(zeros_v, deg_s.at[pl.ds(rowbase, RPT)])
    plsc.subcore_barrier()

    @pl.loop(0, CPT)
    def _(j):
        pltpu.sync_copy(ones_v, deg_s.at[idx_d.at[j]], add=True)
    plsc.subcore_barrier()

    pltpu.sync_copy(deg_s.at[pl.ds(rowbase, RPT)], D)

    @pl.loop(0, RPT)
    def _(i):
        D[i] = (1.0 - ALPHA) / jnp.maximum(D[i], 1.0)

    for k in range(K_LAYER):
        src2d = z0_hbm if k == 0 else out_hbm.at[c]

        pltpu.async_copy(src2d.at[idx_s.at[0]], rb0, sem0)
        pltpu.async_copy(src2d.at[idx_s.at[1]], rb1, sem1)

        @pl.loop(0, CPT // 2)
        def _(t):
            j0 = 2 * t
            pltpu.make_async_copy(src2d.at[idx_s.at[j0]], rb0, sem0).wait()
            pltpu.sync_copy(rb0, agg_s.at[idx_d.at[j0]], add=True)

            @pl.when(j0 + 2 < CPT)
            def _():
                pltpu.async_copy(src2d.at[idx_s.at[j0 + 2]], rb0, sem0)

            pltpu.make_async_copy(src2d.at[idx_s.at[j0 + 1]], rb1, sem1).wait()
            pltpu.sync_copy(rb1, agg_s.at[idx_d.at[j0 + 1]], add=True)

            @pl.when(j0 + 3 < CPT)
            def _():
                pltpu.async_copy(src2d.at[idx_s.at[j0 + 3]], rb1, sem1)

        plsc.subcore_barrier()

        pltpu.sync_copy(agg_s.at[pl.ds(rowbase, RPT)], A)

        @pl.loop(0, RPT)
        def _(i):
            A[i] = A[i] * D[i] + ALPHA * Z[i]

        pltpu.sync_copy(A, out_hbm.at[c, pl.ds(rowbase, RPT)])
        if k < K_LAYER - 1:
            pltpu.sync_copy(zeros_v, agg_s.at[pl.ds(rowbase, RPT)])
        plsc.subcore_barrier()


_propagate = pl.kernel(
    _prop_body,
    out_type=jax.ShapeDtypeStruct((NC, N_PAD, LABELS), jnp.float32),
    mesh=plsc.VectorSubcoreMesh(core_axis_name="c", subcore_axis_name="s"),
    compiler_params=pltpu.CompilerParams(use_tc_tiling_on_sc=False),
    scratch_types=[
        pltpu.VMEM((CPT, CHUNK), jnp.int32),
        pltpu.VMEM((CPT, CHUNK), jnp.int32),
        pltpu.VMEM((CHUNK, LABELS), jnp.float32),
        pltpu.VMEM((CHUNK, LABELS), jnp.float32),
        pltpu.VMEM((CHUNK, LABELS), jnp.float32),
        pltpu.VMEM((RPT, LABELS), jnp.float32),
        pltpu.VMEM((RPT, LABELS), jnp.float32),
        pltpu.VMEM((RPT, LABELS), jnp.float32),
        pltpu.VMEM((RPT, LABELS), jnp.float32),
        pltpu.VMEM_SHARED((N_PAD + 8, LABELS), jnp.float32),
        pltpu.VMEM_SHARED((N_PAD + 8, LABELS), jnp.float32),
        pltpu.SemaphoreType.DMA,
        pltpu.SemaphoreType.DMA,
    ],
)


def _pad_edges(ei):
    src = jnp.concatenate([ei[0], jnp.zeros((E_PAD - E,), jnp.int32)])
    dst = jnp.concatenate([ei[1], jnp.full((E_PAD - E,), DUMP, jnp.int32)])
    return src.reshape(NS * CPT, CHUNK), dst.reshape(NS * CPT, CHUNK)


def _att_body(h_ref, watt_ref, batt_ref, q_ref, o_ref):
    h0 = h_ref[0]
    h1 = h_ref[1]
    a0 = jnp.tanh(jnp.dot(h0, watt_ref[...], preferred_element_type=jnp.float32)
                  + batt_ref[...])
    a1 = jnp.tanh(jnp.dot(h1, watt_ref[...], preferred_element_type=jnp.float32)
                  + batt_ref[...])
    w0 = jnp.sum(a0 * q_ref[...]) / N
    w1 = jnp.sum(a1 * q_ref[...]) / N
    m = jnp.maximum(w0, w1)
    e0 = jnp.exp(w0 - m)
    e1 = jnp.exp(w1 - m)
    inv = 1.0 / (e0 + e1)
    o_ref[...] = (e0 * inv) * h0 + (e1 * inv) * h1


def _fuse(H, W_att, b_att, q_att):
    return pl.pallas_call(
        _att_body,
        out_shape=jax.ShapeDtypeStruct((N, LABELS), jnp.float32),
    )(H, W_att, b_att.reshape(1, ATT), q_att.reshape(1, ATT))


def kernel(x, aug_feats, edge_index_1, edge_index_2, W_proj, b_proj,
           W_att, b_att, q_att):
    z0 = _project(x, aug_feats, W_proj, b_proj)
    s1, d1 = _pad_edges(edge_index_1)
    s2, d2 = _pad_edges(edge_index_2)
    srcs = jnp.stack([s1, s2])
    dsts = jnp.stack([d1, d2])
    H = _propagate(z0, srcs, dsts)
    return _fuse(H[:, :N], W_att, b_att, q_att)


# speedup vs baseline: 16.4419x; 1.1807x over previous
import functools

import jax
import jax.numpy as jnp
from jax import lax
from jax.experimental import pallas as pl
from jax.experimental.pallas import tpu as pltpu
from jax.experimental.pallas import tpu_sc as plsc

N = 10000
E = 320000
D_FEAT = 128
EMB = 64
LABELS = 16
ATT = 128
K_LAYER = 3
ALPHA = 0.1

NC = 2
NS = 16
L = 16

CHUNK = 128
CPT = 160
E_PAD = CPT * CHUNK * NS
N_PAD = 10240
RPT = N_PAD // NS
DUMP = N_PAD


def _proj_body(x_ref, aug_ref, wt_ref, wb_ref, b_ref, o_ref):
    temp = (aug_ref[0] + aug_ref[1] + aug_ref[2] + aug_ref[3]) * 0.25
    z = jnp.dot(x_ref[...], wt_ref[...], preferred_element_type=jnp.float32)
    z = z + jnp.dot(temp, wb_ref[...], preferred_element_type=jnp.float32)
    z = z + b_ref[...]
    o_ref[pl.ds(0, N)] = jnp.maximum(z, 0.0)
    o_ref[pl.ds(N, N_PAD - N)] = jnp.zeros((N_PAD - N, LABELS), jnp.float32)


def _project(x, aug_feats, W_proj, b_proj):
    return pl.pallas_call(
        _proj_body,
        out_shape=jax.ShapeDtypeStruct((N_PAD, LABELS), jnp.float32),
    )(x, aug_feats, W_proj[:D_FEAT], W_proj[D_FEAT:], b_proj.reshape(1, LABELS))


def _prop_body(z0_hbm, srcs_hbm, dsts_hbm, out_hbm,
               idx_s, idx_d, rb0, rb1, rb2, rb3, ones_v, A, D, Z, zeros_v,
               agg_s, deg_s, sg0, sg1, sg2, sg3, ss0, ss1, ss2, ss3, sem_d):
    rbs = (rb0, rb1, rb2, rb3)
    sem_g = (sg0, sg1, sg2, sg3)
    sem_s = (ss0, ss1, ss2, ss3)
    c = lax.axis_index("c")
    s = lax.axis_index("s")
    rowbase = s * RPT

    pltpu.sync_copy(srcs_hbm.at[c, pl.ds(s * CPT, CPT)], idx_s)
    pltpu.sync_copy(dsts_hbm.at[c, pl.ds(s * CPT, CPT)], idx_d)

    @pl.loop(0, CHUNK)
    def _(i):
        ones_v[i] = jnp.full((L,), 1.0, jnp.float32)

    @pl.loop(0, RPT)
    def _(i):
        zeros_v[i] = jnp.zeros((L,), jnp.float32)

    pltpu.sync_copy(z0_hbm.at[pl.ds(rowbase, RPT)], Z)

    pltpu.sync_copy(zeros_v, agg_s.at[pl.ds(rowbase, RPT)])
    pltpu.sync_copy(zeros_v, deg_s.at[pl.ds(rowbase, RPT)])
    plsc.subcore_barrier()

    NB = len(rbs)

    for k in range(K_LAYER):
        src2d = z0_hbm if k == 0 else out_hbm.at[c]

        for b in range(NB - 1):
            pltpu.async_copy(src2d.at[idx_s.at[b]], rbs[b], sem_g[b])

        @pl.loop(0, CPT // NB)
        def _(t):
            for b in range(NB):
                j = NB * t + b
                bp = (b - 1) % NB
                pltpu.make_async_copy(src2d.at[idx_s.at[j]], rbs[b],
                                      sem_g[b]).wait()
                pltpu.async_copy(rbs[b], agg_s.at[idx_d.at[j]], sem_s[b],
                                 add=True)
                if k == 0:
                    pltpu.async_copy(ones_v, deg_s.at[idx_d.at[j]], sem_d,
                                     add=True)

                @pl.when(j + NB - 1 < CPT)
                def _():
                    @pl.when(j > 0)
                    def _():
                        pltpu.make_async_copy(
                            rbs[bp], agg_s.at[idx_d.at[0]], sem_s[bp]).wait()
                    pltpu.async_copy(src2d.at[idx_s.at[j + NB - 1]], rbs[bp],
                                     sem_g[bp])

        for b in range(NB):
            pltpu.make_async_copy(rbs[b], agg_s.at[idx_d.at[0]],
                                  sem_s[b]).wait()
        if k == 0:
            @pl.loop(0, CPT)
            def _(j):
                pltpu.make_async_copy(ones_v, deg_s.at[idx_d.at[0]],
                                      sem_d).wait()
        plsc.subcore_barrier()

        if k == 0:
            pltpu.sync_copy(deg_s.at[pl.ds(rowbase, RPT)], D)

            @pl.loop(0, RPT)
            def _(i):
                D[i] = (1.0 - ALPHA) / jnp.maximum(D[i], 1.0)

        pltpu.sync_copy(agg_s.at[pl.ds(rowbase, RPT)], A)

        @pl.loop(0, RPT)
        def _(i):
            A[i] = A[i] * D[i] + ALPHA * Z[i]

        pltpu.sync_copy(A, out_hbm.at[c, pl.ds(rowbase, RPT)])
        if k < K_LAYER - 1:
            pltpu.sync_copy(zeros_v, agg_s.at[pl.ds(rowbase, RPT)])
        plsc.subcore_barrier()


_propagate = pl.kernel(
    _prop_body,
    out_type=jax.ShapeDtypeStruct((NC, N_PAD, LABELS), jnp.float32),
    mesh=plsc.VectorSubcoreMesh(core_axis_name="c", subcore_axis_name="s"),
    compiler_params=pltpu.CompilerParams(use_tc_tiling_on_sc=False),
    scratch_types=[
        pltpu.VMEM((CPT, CHUNK), jnp.int32),
        pltpu.VMEM((CPT, CHUNK), jnp.int32),
        pltpu.VMEM((CHUNK, LABELS), jnp.float32),
        pltpu.VMEM((CHUNK, LABELS), jnp.float32),
        pltpu.VMEM((CHUNK, LABELS), jnp.float32),
        pltpu.VMEM((CHUNK, LABELS), jnp.float32),
        pltpu.VMEM((CHUNK, LABELS), jnp.float32),
        pltpu.VMEM((RPT, LABELS), jnp.float32),
        pltpu.VMEM((RPT, LABELS), jnp.float32),
        pltpu.VMEM((RPT, LABELS), jnp.float32),
        pltpu.VMEM((RPT, LABELS), jnp.float32),
        pltpu.VMEM_SHARED((N_PAD + 8, LABELS), jnp.float32),
        pltpu.VMEM_SHARED((N_PAD + 8, LABELS), jnp.float32),
        pltpu.SemaphoreType.DMA,
        pltpu.SemaphoreType.DMA,
        pltpu.SemaphoreType.DMA,
        pltpu.SemaphoreType.DMA,
        pltpu.SemaphoreType.DMA,
        pltpu.SemaphoreType.DMA,
        pltpu.SemaphoreType.DMA,
        pltpu.SemaphoreType.DMA,
        pltpu.SemaphoreType.DMA,
    ],
)


def _pad_edges(ei):
    src = jnp.concatenate([ei[0], jnp.zeros((E_PAD - E,), jnp.int32)])
    dst = jnp.concatenate([ei[1], jnp.full((E_PAD - E,), DUMP, jnp.int32)])
    return src.reshape(NS * CPT, CHUNK), dst.reshape(NS * CPT, CHUNK)


def _att_body(h_ref, watt_ref, batt_ref, q_ref, o_ref):
    h0 = h_ref[0]
    h1 = h_ref[1]
    a0 = jnp.tanh(jnp.dot(h0, watt_ref[...], preferred_element_type=jnp.float32)
                  + batt_ref[...])
    a1 = jnp.tanh(jnp.dot(h1, watt_ref[...], preferred_element_type=jnp.float32)
                  + batt_ref[...])
    w0 = jnp.sum(a0 * q_ref[...]) / N
    w1 = jnp.sum(a1 * q_ref[...]) / N
    m = jnp.maximum(w0, w1)
    e0 = jnp.exp(w0 - m)
    e1 = jnp.exp(w1 - m)
    inv = 1.0 / (e0 + e1)
    o_ref[...] = (e0 * inv) * h0 + (e1 * inv) * h1


def _fuse(H, W_att, b_att, q_att):
    return pl.pallas_call(
        _att_body,
        out_shape=jax.ShapeDtypeStruct((N, LABELS), jnp.float32),
    )(H, W_att, b_att.reshape(1, ATT), q_att.reshape(1, ATT))


def kernel(x, aug_feats, edge_index_1, edge_index_2, W_proj, b_proj,
           W_att, b_att, q_att):
    z0 = _project(x, aug_feats, W_proj, b_proj)
    s1, d1 = _pad_edges(edge_index_1)
    s2, d2 = _pad_edges(edge_index_2)
    srcs = jnp.stack([s1, s2])
    dsts = jnp.stack([d1, d2])
    H = _propagate(z0, srcs, dsts)
    return _fuse(H[:, :N], W_att, b_att, q_att)


# speedup vs baseline: 26.0793x; 1.5862x over previous
import functools

import jax
import jax.numpy as jnp
from jax import lax
from jax.experimental import pallas as pl
from jax.experimental.pallas import tpu as pltpu
from jax.experimental.pallas import tpu_sc as plsc

N = 10000
E = 320000
D_FEAT = 128
EMB = 64
LABELS = 16
ATT = 128
K_LAYER = 3
ALPHA = 0.1

NC = 2
NS = 16
L = 16

CHUNK = 128
CPT = 160
E_PAD = CPT * CHUNK * NS
N_PAD = 10240
RPT = N_PAD // NS
DUMP = N_PAD


def _proj_body(x_ref, aug_ref, wt_ref, wb_ref, b_ref, o_ref):
    temp = (aug_ref[0] + aug_ref[1] + aug_ref[2] + aug_ref[3]) * 0.25
    z = jnp.dot(x_ref[...], wt_ref[...], preferred_element_type=jnp.float32)
    z = z + jnp.dot(temp, wb_ref[...], preferred_element_type=jnp.float32)
    z = z + b_ref[...]
    o_ref[pl.ds(0, N)] = jnp.maximum(z, 0.0)
    o_ref[pl.ds(N, N_PAD - N)] = jnp.zeros((N_PAD - N, LABELS), jnp.float32)


def _project(x, aug_feats, W_proj, b_proj):
    return pl.pallas_call(
        _proj_body,
        out_shape=jax.ShapeDtypeStruct((N_PAD, LABELS), jnp.float32),
    )(x, aug_feats, W_proj[:D_FEAT], W_proj[D_FEAT:], b_proj.reshape(1, LABELS))


def _prop_body(z0_hbm, srcs_hbm, dsts_hbm, out_hbm,
               idx_s, idx_d, rb0, rb1, rb2, rb3, ones_v, A, D, Z, zeros_v,
               h_s, agg_s, deg_s, sg0, sg1, sg2, sg3, ss0, ss1, ss2, ss3, sem_d):
    rbs = (rb0, rb1, rb2, rb3)
    sem_g = (sg0, sg1, sg2, sg3)
    sem_s = (ss0, ss1, ss2, ss3)
    c = lax.axis_index("c")
    s = lax.axis_index("s")
    rowbase = s * RPT

    pltpu.sync_copy(srcs_hbm.at[c, pl.ds(s * CPT, CPT)], idx_s)
    pltpu.sync_copy(dsts_hbm.at[c, pl.ds(s * CPT, CPT)], idx_d)

    @pl.loop(0, CHUNK)
    def _(i):
        ones_v[i] = jnp.full((L,), 1.0, jnp.float32)

    @pl.loop(0, RPT)
    def _(i):
        zeros_v[i] = jnp.zeros((L,), jnp.float32)

    pltpu.sync_copy(z0_hbm.at[pl.ds(rowbase, RPT)], Z)
    pltpu.sync_copy(z0_hbm.at[pl.ds(rowbase, RPT)], h_s.at[pl.ds(rowbase, RPT)])

    pltpu.sync_copy(zeros_v, agg_s.at[pl.ds(rowbase, RPT)])
    pltpu.sync_copy(zeros_v, deg_s.at[pl.ds(rowbase, RPT)])
    plsc.subcore_barrier()

    NB = len(rbs)

    for k in range(K_LAYER):
        src2d = h_s

        for b in range(NB - 1):
            pltpu.async_copy(src2d.at[idx_s.at[b]], rbs[b], sem_g[b])

        @pl.loop(0, CPT // NB)
        def _(t):
            for b in range(NB):
                j = NB * t + b
                bp = (b - 1) % NB
                pltpu.make_async_copy(src2d.at[idx_s.at[j]], rbs[b],
                                      sem_g[b]).wait()
                pltpu.async_copy(rbs[b], agg_s.at[idx_d.at[j]], sem_s[b],
                                 add=True)
                if k == 0:
                    pltpu.async_copy(ones_v, deg_s.at[idx_d.at[j]], sem_d,
                                     add=True)

                @pl.when(j + NB - 1 < CPT)
                def _():
                    @pl.when(j > 0)
                    def _():
                        pltpu.make_async_copy(
                            rbs[bp], agg_s.at[idx_d.at[0]], sem_s[bp]).wait()
                    pltpu.async_copy(src2d.at[idx_s.at[j + NB - 1]], rbs[bp],
                                     sem_g[bp])

        for b in range(NB):
            pltpu.make_async_copy(rbs[b], agg_s.at[idx_d.at[0]],
                                  sem_s[b]).wait()
        if k == 0:
            @pl.loop(0, CPT)
            def _(j):
                pltpu.make_async_copy(ones_v, deg_s.at[idx_d.at[0]],
                                      sem_d).wait()
        plsc.subcore_barrier()

        if k == 0:
            pltpu.sync_copy(deg_s.at[pl.ds(rowbase, RPT)], D)

            @pl.loop(0, RPT)
            def _(i):
                D[i] = (1.0 - ALPHA) / jnp.maximum(D[i], 1.0)

        pltpu.sync_copy(agg_s.at[pl.ds(rowbase, RPT)], A)

        @pl.loop(0, RPT)
        def _(i):
            A[i] = A[i] * D[i] + ALPHA * Z[i]

        if k < K_LAYER - 1:
            pltpu.sync_copy(A, h_s.at[pl.ds(rowbase, RPT)])
            pltpu.sync_copy(zeros_v, agg_s.at[pl.ds(rowbase, RPT)])
        else:
            pltpu.sync_copy(A, out_hbm.at[c, pl.ds(rowbase, RPT)])
        plsc.subcore_barrier()


_propagate = pl.kernel(
    _prop_body,
    out_type=jax.ShapeDtypeStruct((NC, N_PAD, LABELS), jnp.float32),
    mesh=plsc.VectorSubcoreMesh(core_axis_name="c", subcore_axis_name="s"),
    compiler_params=pltpu.CompilerParams(use_tc_tiling_on_sc=False),
    scratch_types=[
        pltpu.VMEM((CPT, CHUNK), jnp.int32),
        pltpu.VMEM((CPT, CHUNK), jnp.int32),
        pltpu.VMEM((CHUNK, LABELS), jnp.float32),
        pltpu.VMEM((CHUNK, LABELS), jnp.float32),
        pltpu.VMEM((CHUNK, LABELS), jnp.float32),
        pltpu.VMEM((CHUNK, LABELS), jnp.float32),
        pltpu.VMEM((CHUNK, LABELS), jnp.float32),
        pltpu.VMEM((RPT, LABELS), jnp.float32),
        pltpu.VMEM((RPT, LABELS), jnp.float32),
        pltpu.VMEM((RPT, LABELS), jnp.float32),
        pltpu.VMEM((RPT, LABELS), jnp.float32),
        pltpu.VMEM_SHARED((N_PAD + 8, LABELS), jnp.float32),
        pltpu.VMEM_SHARED((N_PAD + 8, LABELS), jnp.float32),
        pltpu.VMEM_SHARED((N_PAD + 8, LABELS), jnp.float32),
        pltpu.SemaphoreType.DMA,
        pltpu.SemaphoreType.DMA,
        pltpu.SemaphoreType.DMA,
        pltpu.SemaphoreType.DMA,
        pltpu.SemaphoreType.DMA,
        pltpu.SemaphoreType.DMA,
        pltpu.SemaphoreType.DMA,
        pltpu.SemaphoreType.DMA,
        pltpu.SemaphoreType.DMA,
    ],
)


def _pad_edges(ei):
    src = jnp.concatenate([ei[0], jnp.zeros((E_PAD - E,), jnp.int32)])
    dst = jnp.concatenate([ei[1], jnp.full((E_PAD - E,), DUMP, jnp.int32)])
    return src.reshape(NS * CPT, CHUNK), dst.reshape(NS * CPT, CHUNK)


def _att_body(h_ref, watt_ref, batt_ref, q_ref, o_ref):
    h0 = h_ref[0]
    h1 = h_ref[1]
    a0 = jnp.tanh(jnp.dot(h0, watt_ref[...], preferred_element_type=jnp.float32)
                  + batt_ref[...])
    a1 = jnp.tanh(jnp.dot(h1, watt_ref[...], preferred_element_type=jnp.float32)
                  + batt_ref[...])
    w0 = jnp.sum(a0 * q_ref[...]) / N
    w1 = jnp.sum(a1 * q_ref[...]) / N
    m = jnp.maximum(w0, w1)
    e0 = jnp.exp(w0 - m)
    e1 = jnp.exp(w1 - m)
    inv = 1.0 / (e0 + e1)
    o_ref[...] = (e0 * inv) * h0 + (e1 * inv) * h1


def _fuse(H, W_att, b_att, q_att):
    return pl.pallas_call(
        _att_body,
        out_shape=jax.ShapeDtypeStruct((N, LABELS), jnp.float32),
    )(H, W_att, b_att.reshape(1, ATT), q_att.reshape(1, ATT))


def kernel(x, aug_feats, edge_index_1, edge_index_2, W_proj, b_proj,
           W_att, b_att, q_att):
    z0 = _project(x, aug_feats, W_proj, b_proj)
    s1, d1 = _pad_edges(edge_index_1)
    s2, d2 = _pad_edges(edge_index_2)
    srcs = jnp.stack([s1, s2])
    dsts = jnp.stack([d1, d2])
    H = _propagate(z0, srcs, dsts)
    return _fuse(H[:, :N], W_att, b_att, q_att)
